# baseline (speedup 1.0000x reference)
import jax
import jax.numpy as jnp
from jax.experimental import pallas as pl
from jax.experimental.pallas import tpu as pltpu

_SIZE = 20


def _shl(a, k):
    return jnp.concatenate(
        [a[:, k:], jnp.zeros((a.shape[0], k), a.dtype)], axis=1
    )


def _ts_mean_kernel(x_ref, o_ref):
    x = x_ref[0]
    s2 = x + _shl(x, 1)
    s4 = s2 + _shl(s2, 2)
    s8 = s4 + _shl(s4, 4)
    s16 = s8 + _shl(s8, 8)
    s20 = s16 + _shl(s4, 16)
    t_out = o_ref.shape[2]
    o_ref[0] = s20[:, :t_out] * (1.0 / _SIZE)


def kernel(x):
    b, f, t = x.shape
    t_out = t - _SIZE + 1
    return pl.pallas_call(
        _ts_mean_kernel,
        grid=(b,),
        in_specs=[pl.BlockSpec((1, f, t), lambda i: (i, 0, 0))],
        out_specs=pl.BlockSpec((1, f, t_out), lambda i: (i, 0, 0)),
        out_shape=jax.ShapeDtypeStruct((b, f, t_out), x.dtype),
        compiler_params=pltpu.CompilerParams(
            dimension_semantics=("parallel",),
        ),
    )(x)


# speedup vs baseline: 1.0018x; 1.0018x over previous
import jax
import jax.numpy as jnp
from jax.experimental import pallas as pl
from jax.experimental.pallas import tpu as pltpu

_SIZE = 20


def _shl(a, k):
    return jnp.concatenate(
        [a[:, k:], jnp.zeros((a.shape[0], k), a.dtype)], axis=1
    )


_BB = 2


def _ts_mean_kernel(x_ref, o_ref):
    for i in range(_BB):
        x = x_ref[i]
        s2 = x + _shl(x, 1)
        s4 = s2 + _shl(s2, 2)
        s8 = s4 + _shl(s4, 4)
        s16 = s8 + _shl(s8, 8)
        s20 = s16 + _shl(s4, 16)
        t_out = o_ref.shape[2]
        o_ref[i] = s20[:, :t_out] * (1.0 / _SIZE)


def kernel(x):
    b, f, t = x.shape
    t_out = t - _SIZE + 1
    return pl.pallas_call(
        _ts_mean_kernel,
        grid=(b // _BB,),
        in_specs=[pl.BlockSpec((_BB, f, t), lambda i: (i, 0, 0))],
        out_specs=pl.BlockSpec((_BB, f, t_out), lambda i: (i, 0, 0)),
        out_shape=jax.ShapeDtypeStruct((b, f, t_out), x.dtype),
        compiler_params=pltpu.CompilerParams(
            dimension_semantics=("parallel",),
        ),
    )(x)


# speedup vs baseline: 1.0804x; 1.0784x over previous
import jax
import jax.numpy as jnp
from jax.experimental import pallas as pl
from jax.experimental.pallas import tpu as pltpu

_SIZE = 20


def _shl(a, k):
    return jnp.concatenate(
        [a[:, k:], jnp.zeros((a.shape[0], k), a.dtype)], axis=1
    )


_BB = 2


def _ts_mean_kernel(x_ref, o_ref):
    for i in range(_BB):
        x = x_ref[i]
        s2 = x + _shl(x, 1)
        s4 = s2 + _shl(s2, 2)
        s8 = s4 + _shl(s4, 4)
        s16 = s8 + _shl(s8, 8)
        s20 = s16 + _shl(s4, 16)
        o_ref[i] = s20 * (1.0 / _SIZE)


def kernel(x):
    b, f, t = x.shape
    t_out = t - _SIZE + 1
    y = pl.pallas_call(
        _ts_mean_kernel,
        grid=(b // _BB,),
        in_specs=[pl.BlockSpec((_BB, f, t), lambda i: (i, 0, 0))],
        out_specs=pl.BlockSpec((_BB, f, t), lambda i: (i, 0, 0)),
        out_shape=jax.ShapeDtypeStruct((b, f, t), x.dtype),
        compiler_params=pltpu.CompilerParams(
            dimension_semantics=("parallel",),
        ),
    )(x)
    return y[..., :t_out]
